# baseline (speedup 1.0000x reference)
import jax
import jax.numpy as jnp
from jax.experimental import pallas as pl

_TAU = 1.0
_ROWS_PER_BLOCK = 8


def _body(x_ref, w_ref, u_ref, o_ref):
    x = x_ref[...]
    w = w_ref[...]
    u = u_ref[...]
    g = -jnp.log(-jnp.log(u))
    l = (x * w + g) * (1.0 / _TAU)
    m = jnp.max(l, axis=-1, keepdims=True)
    e = jnp.exp(l - m)
    s = jnp.sum(e, axis=-1, keepdims=True)
    o_ref[...] = e / s


def kernel(x, weights):
    b, n = x.shape
    u = jax.random.uniform(jax.random.key(42), (b, n), dtype=jnp.float32,
                           minval=1e-20, maxval=1.0)
    grid = b // _ROWS_PER_BLOCK
    return pl.pallas_call(
        _body,
        grid=(grid,),
        in_specs=[
            pl.BlockSpec((_ROWS_PER_BLOCK, n), lambda i: (i, 0)),
            pl.BlockSpec((1, n), lambda i: (0, 0)),
            pl.BlockSpec((_ROWS_PER_BLOCK, n), lambda i: (i, 0)),
        ],
        out_specs=pl.BlockSpec((_ROWS_PER_BLOCK, n), lambda i: (i, 0)),
        out_shape=jax.ShapeDtypeStruct((b, n), jnp.float32),
    )(x, weights, u)
